# Initial kernel scaffold; baseline (speedup 1.0000x reference)
import jax
import jax.numpy as jnp
from jax.experimental import pallas as pl


def kernel(data, edge_index, W_rel, b_rel, W_root, Ws_rel, bs_rel, Ws_root, W1, b1, W2, b2, W3, b3):
    raise NotImplementedError("write your pallas kernel here")


# speedup vs baseline: 3.1964x; 3.1964x over previous
import functools

import jax
import jax.numpy as jnp
from jax import lax
from jax.experimental import pallas as pl
from jax.experimental.pallas import tpu as pltpu
from jax.experimental.pallas import tpu_sc as plsc

N = 10000
F_IN = 128
H = 16
B = 2
E = 160000
K = 4000

NT = 16
NR = N // NT
CAP = 16384
CAPP = CAP + 256
CH = 256
SLAB_F = 800
KP = 4096
KPT = KP // 16

_mesh = plsc.VectorSubcoreMesh(core_axis_name="c", subcore_axis_name="s")
_sc_params = pltpu.CompilerParams(use_tc_tiling_on_sc=False)


EROWS = E // 128


def _route_body(src_ref, dst_ref, tgt_ref, pk_ref, cnt_ref):
    dst = dst_ref[...]
    src = src_ref[...]
    rid = dst // NR
    dl = dst - rid * NR
    wrow = jnp.zeros_like(dst)
    rows = []
    l0 = lax.broadcasted_iota(jnp.int32, (128, 128), 0)
    l1 = lax.broadcasted_iota(jnp.int32, (128, 128), 1)
    ltl = (l0 < l1).astype(jnp.float32)
    for r in range(NT):
        mr = rid == r
        mrf = mr.astype(jnp.float32)
        cr = lax.dot_general(mrf, ltl, (((1,), (0,)), ((), ())),
                             preferred_element_type=jnp.float32)
        wrow = wrow + jnp.where(mr, cr.astype(jnp.int32), 0)
        rows.append(jnp.sum(mrf, axis=1, keepdims=True))
    rowsum = jnp.concatenate(rows, axis=1).astype(jnp.float32)
    i0 = lax.broadcasted_iota(jnp.int32, (EROWS, EROWS), 0)
    i1 = lax.broadcasted_iota(jnp.int32, (EROWS, EROWS), 1)
    lt = (i1 < i0).astype(jnp.float32)
    rowpref = lax.dot_general(lt, rowsum, (((1,), (0,)), ((), ())),
                              preferred_element_type=jnp.float32)
    base = jnp.take_along_axis(rowpref.astype(jnp.int32), rid, axis=1)
    pos = base + wrow
    tgt_ref[...] = rid * CAPP + pos
    pk_ref[...] = src * 1024 + dl
    cnt_ref[...] = (rowpref + rowsum).astype(jnp.int32)


def _route(src2d, dst2d):
    return pl.pallas_call(
        _route_body,
        out_shape=[
            jax.ShapeDtypeStruct((EROWS, 128), jnp.int32),
            jax.ShapeDtypeStruct((EROWS, 128), jnp.int32),
            jax.ShapeDtypeStruct((EROWS, NT), jnp.int32),
        ],
    )(src2d, dst2d)


EPT = E // NT
CHS = 80


@functools.partial(
    pl.kernel,
    out_type=jax.ShapeDtypeStruct((B * NT * CAPP,), jnp.int32),
    mesh=_mesh,
    compiler_params=_sc_params,
    scratch_types=[
        pltpu.VMEM((CHS,), jnp.int32),
        pltpu.VMEM((CHS,), jnp.int32),
        pltpu.VMEM((CAPP,), jnp.int32),
        pltpu.VMEM_SHARED((NT * CAPP,), jnp.int32),
        pltpu.SemaphoreType.DMA,
    ],
)
def _scatlist(tgt_hbm, pk_hbm, lists_hbm, tgtv, pkv, fill, listsS, sem):
    c = lax.axis_index("c")
    t = lax.axis_index("s")

    dummy = jnp.full((16,), c * N * 1024 + NR, jnp.int32)

    def fl(i, _):
        fill[pl.ds(i * 16, 16)] = dummy
        return 0

    lax.fori_loop(0, CAPP // 16, fl, 0)
    pltpu.sync_copy(fill, listsS.at[pl.ds(t * CAPP, CAPP)])
    plsc.subcore_barrier()

    def chunk(j, _):
        pltpu.sync_copy(tgt_hbm.at[pl.ds(t * EPT + j * CHS, CHS)], tgtv)
        pltpu.sync_copy(pk_hbm.at[pl.ds(c * E + t * EPT + j * CHS, CHS)], pkv)
        pltpu.sync_copy(pkv, listsS.at[tgtv])
        return 0

    lax.fori_loop(0, EPT // CHS, chunk, 0)
    plsc.subcore_barrier()
    pltpu.sync_copy(listsS.at[pl.ds(t * CAPP, CAPP)],
                    lists_hbm.at[pl.ds((c * NT + t) * CAPP, CAPP)])


def _make_segsum(width):
    @functools.partial(
        pl.kernel,
        out_type=jax.ShapeDtypeStruct((B * N, width), jnp.float32),
        mesh=_mesh,
        compiler_params=_sc_params,
        scratch_types=[
            pltpu.VMEM((CH,), jnp.int32),
            pltpu.VMEM((CH,), jnp.int32),
            pltpu.VMEM((CH, width), jnp.float32),
            pltpu.VMEM((16,), jnp.int32),
            pltpu.VMEM((NR + 16, width), jnp.float32),
            pltpu.SemaphoreType.DMA,
        ],
    )
    def seg(lists_hbm, cnt_hbm, table_hbm, out_hbm,
            pkv, idxv, rows, cntv, acc, sem):
        c = lax.axis_index("c")
        t = lax.axis_index("s")
        base = (c * NT + t) * CAPP

        zero = jnp.zeros((16,), jnp.float32)

        def zr(i, _):
            for v in range(width // 16):
                acc[i, pl.ds(v * 16, 16)] = zero
            return 0

        lax.fori_loop(0, NR + 16, zr, 0)

        pltpu.sync_copy(cnt_hbm.at[pl.ds((c * NT + t) * 16, 16)], cntv)
        nrounds = cntv[pl.ds(0, 16)][0]

        def rnd(r, _):
            pltpu.sync_copy(lists_hbm.at[pl.ds(base + r * CH, CH)], pkv)

            def unp(j, _):
                idxv[pl.ds(j * 16, 16)] = (
                    lax.shift_right_logical(pkv[pl.ds(j * 16, 16)], 10))
                return 0

            lax.fori_loop(0, CH // 16, unp, 0)
            pltpu.async_copy(table_hbm.at[idxv], rows, sem).wait()

            def grp(j, _):
                dv = pkv[pl.ds(j * 16, 16)] & 1023
                for k in range(16):
                    d = dv[k]
                    for v in range(width // 16):
                        plsc.addupdate(acc.at[d, pl.ds(v * 16, 16)],
                                       rows[j * 16 + k, pl.ds(v * 16, 16)])
                return 0

            lax.fori_loop(0, CH // 16, grp, 0)
            return 0

        lax.fori_loop(0, nrounds, rnd, 0)
        pltpu.sync_copy(acc.at[pl.ds(0, NR)],
                        out_hbm.at[pl.ds(c * N + t * NR, NR)])

    return seg


_segsum128 = _make_segsum(F_IN)
_segsum16 = _make_segsum(H)


@functools.partial(
    pl.kernel,
    out_type=jax.ShapeDtypeStruct((B, KP, H), jnp.float32),
    mesh=_mesh,
    compiler_params=_sc_params,
    scratch_types=[
        pltpu.VMEM((KPT,), jnp.int32),
        pltpu.VMEM((KPT,), jnp.float32),
        pltpu.VMEM((KPT, H), jnp.float32),
        pltpu.VMEM((KPT, H), jnp.float32),
        pltpu.SemaphoreType.DMA,
    ],
)
def _pool(h_hbm, perm_hbm, tops_hbm, out_hbm, permv, sv, rows, orows, sem):
    c = lax.axis_index("c")
    s = lax.axis_index("s")
    pltpu.sync_copy(perm_hbm.at[c, pl.ds(s * KPT, KPT)], permv)
    pltpu.sync_copy(tops_hbm.at[c, pl.ds(s * KPT, KPT)], sv)
    pltpu.async_copy(h_hbm.at[permv], rows, sem).wait()

    def body(j, carry):
        svc = sv[pl.ds(j * 16, 16)]
        for k in range(16):
            r = j * 16 + k
            orows[r] = rows[r] * svc[k]
        return carry

    lax.fori_loop(0, KPT // 16, body, 0)
    pltpu.sync_copy(orows, out_hbm.at[c, pl.ds(s * KPT, KPT)])


def _hid_body(agg_ref, x_ref, wrel_ref, wroot_ref, brel_ref, h_ref):
    d1 = lax.dot_general(agg_ref[...], wrel_ref[...], (((1,), (1,)), ((), ())),
                         preferred_element_type=jnp.float32)
    t1 = d1 + brel_ref[...]
    d2 = lax.dot_general(x_ref[...], wroot_ref[...], (((1,), (1,)), ((), ())),
                         preferred_element_type=jnp.float32)
    h_ref[...] = jnp.maximum(t1 + d2, 0.0)


def _hid(agg, x, W_rel, W_root, brel):
    m_blk = 4000
    return pl.pallas_call(
        _hid_body,
        grid=(B * N // m_blk,),
        in_specs=[
            pl.BlockSpec((m_blk, F_IN), lambda i: (i, 0)),
            pl.BlockSpec((m_blk, F_IN), lambda i: (i, 0)),
            pl.BlockSpec((H, F_IN), lambda i: (0, 0)),
            pl.BlockSpec((H, F_IN), lambda i: (0, 0)),
            pl.BlockSpec((1, H), lambda i: (0, 0)),
        ],
        out_specs=pl.BlockSpec((m_blk, H), lambda i: (i, 0)),
        out_shape=jax.ShapeDtypeStruct((B * N, H), jnp.float32),
    )(agg, x, W_rel, W_root, brel)


def _score_body(agg2_ref, h_ref, wsrel_ref, wsroot_ref, bs_ref, o_ref):
    d1 = lax.dot_general(agg2_ref[...], wsrel_ref[...], (((1,), (0,)), ((), ())),
                         preferred_element_type=jnp.float32)
    t1 = d1 + bs_ref[0, 0]
    d2 = lax.dot_general(h_ref[...], wsroot_ref[...], (((1,), (0,)), ((), ())),
                         preferred_element_type=jnp.float32)
    o_ref[...] = jnp.tanh(t1 + d2)


def _score(agg2, h, Ws_rel, Ws_root, bs):
    wr = jnp.pad(Ws_rel.T, ((0, 0), (0, 127)))
    wo = jnp.pad(Ws_root.T, ((0, 0), (0, 127)))
    out = pl.pallas_call(
        _score_body,
        out_shape=jax.ShapeDtypeStruct((B * N, 128), jnp.float32),
    )(agg2, h, wr, wo, bs)
    return out[:, 0]


_FC_CHUNK = 6400
_FC_STEPS = (K * H) // _FC_CHUNK


def _fc_body(pf_ref, w1_ref, b1_ref, w2_ref, b2_ref, w3_ref, b3_ref,
             o_ref, acc_ref):
    k = pl.program_id(0)

    @pl.when(k == 0)
    def _():
        acc_ref[...] = jnp.zeros_like(acc_ref)

    acc_ref[...] += lax.dot_general(
        pf_ref[...], w1_ref[...], (((1,), (1,)), ((), ())),
        precision=lax.Precision.HIGHEST,
        preferred_element_type=jnp.float32)

    @pl.when(k == _FC_STEPS - 1)
    def _():
        xf = jnp.maximum(acc_ref[...] + b1_ref[...], 0.0)
        xf = jnp.maximum(
            lax.dot_general(xf, w2_ref[...], (((1,), (1,)), ((), ())),
                            precision=lax.Precision.HIGHEST,
                            preferred_element_type=jnp.float32)
            + b2_ref[...], 0.0)
        o_ref[...] = (jnp.sum(xf * w3_ref[...], axis=1, keepdims=True)
                      + b3_ref[...])


def _fc(pf, W1, b1, W2, b2, W3, b3):
    return pl.pallas_call(
        _fc_body,
        grid=(_FC_STEPS,),
        in_specs=[
            pl.BlockSpec((B, _FC_CHUNK), lambda k: (0, k)),
            pl.BlockSpec((256, _FC_CHUNK), lambda k: (0, k)),
            pl.BlockSpec((1, 256), lambda k: (0, 0)),
            pl.BlockSpec((64, 256), lambda k: (0, 0)),
            pl.BlockSpec((1, 64), lambda k: (0, 0)),
            pl.BlockSpec((1, 64), lambda k: (0, 0)),
            pl.BlockSpec((1, 1), lambda k: (0, 0)),
        ],
        out_specs=pl.BlockSpec((B, 1), lambda k: (0, 0)),
        out_shape=jax.ShapeDtypeStruct((B, 1), jnp.float32),
        scratch_shapes=[pltpu.VMEM((B, 256), jnp.float32)],
    )(pf, W1, b1, W2, b2, W3, b3)


def kernel(data, edge_index, W_rel, b_rel, W_root, Ws_rel, bs_rel, Ws_root,
           W1, b1, W2, b2, W3, b3):
    x = data.reshape(B * N, F_IN)
    src2d = edge_index[0].reshape(EROWS, 128)
    dst2d = edge_index[1].reshape(EROWS, 128)

    tgt, pk, cnt_rows = _route(src2d, dst2d)
    tgt = tgt.reshape(E)
    pk = pk.reshape(E)
    pk2 = jnp.concatenate([pk, pk + N * 1024])
    cnts = cnt_rows[EROWS - 1]
    nrounds = (cnts + CH - 1) // CH
    nr_splat = jnp.broadcast_to(nrounds[None, :, None], (B, NT, 16)).reshape(-1)

    lists = _scatlist(tgt, pk2)
    agg = _segsum128(lists, nr_splat, x)
    h = _hid(agg, x, W_rel, W_root, b_rel.reshape(1, H))
    agg2 = _segsum16(lists, nr_splat, h)
    score = _score(agg2, h, Ws_rel, Ws_root, bs_rel.reshape(1, 1))
    score = score.reshape(B, N)

    top_s, perm = lax.top_k(score, K)
    offs = jnp.arange(B, dtype=jnp.int32) * N
    perm_g = jnp.pad(perm + offs[:, None], ((0, 0), (0, KP - K)))
    tops_p = jnp.pad(top_s, ((0, 0), (0, KP - K)))

    pooled = _pool(h, perm_g, tops_p)
    pf = pooled[:, :K, :].reshape(B, K * H)

    return _fc(pf, W1, b1.reshape(1, 256), W2, b2.reshape(1, 64),
               W3, b3.reshape(1, 1))


# speedup vs baseline: 3.6844x; 1.1527x over previous
import functools

import jax
import jax.numpy as jnp
from jax import lax
from jax.experimental import pallas as pl
from jax.experimental.pallas import tpu as pltpu
from jax.experimental.pallas import tpu_sc as plsc

N = 10000
F_IN = 128
H = 16
B = 2
E = 160000
K = 4000

NT = 16
NR = N // NT
CAP = 16384
CAPP = CAP + 256
CH = 256
SLAB_F = 800
KP = 4096
KPT = KP // 16

_mesh = plsc.VectorSubcoreMesh(core_axis_name="c", subcore_axis_name="s")
_sc_params = pltpu.CompilerParams(use_tc_tiling_on_sc=False)


EROWS = E // 128


def _route_body(src_ref, dst_ref, tgt_ref, pk_ref, cnt_ref):
    dst = dst_ref[...]
    src = src_ref[...]
    rid = dst // NR
    dl = dst - rid * NR
    wrow = jnp.zeros_like(dst)
    rows = []
    l0 = lax.broadcasted_iota(jnp.int32, (128, 128), 0)
    l1 = lax.broadcasted_iota(jnp.int32, (128, 128), 1)
    ltl = (l0 < l1).astype(jnp.float32)
    for r in range(NT):
        mr = rid == r
        mrf = mr.astype(jnp.float32)
        cr = lax.dot_general(mrf, ltl, (((1,), (0,)), ((), ())),
                             preferred_element_type=jnp.float32)
        wrow = wrow + jnp.where(mr, cr.astype(jnp.int32), 0)
        rows.append(jnp.sum(mrf, axis=1, keepdims=True))
    rowsum = jnp.concatenate(rows, axis=1).astype(jnp.float32)
    i0 = lax.broadcasted_iota(jnp.int32, (EROWS, EROWS), 0)
    i1 = lax.broadcasted_iota(jnp.int32, (EROWS, EROWS), 1)
    lt = (i1 < i0).astype(jnp.float32)
    rowpref = lax.dot_general(lt, rowsum, (((1,), (0,)), ((), ())),
                              preferred_element_type=jnp.float32)
    base = jnp.take_along_axis(rowpref.astype(jnp.int32), rid, axis=1)
    pos = base + wrow
    tgt_ref[...] = rid * CAPP + pos
    pk_ref[...] = src * 1024 + dl
    cnt_ref[...] = (rowpref + rowsum).astype(jnp.int32)


def _route(src2d, dst2d):
    return pl.pallas_call(
        _route_body,
        out_shape=[
            jax.ShapeDtypeStruct((EROWS, 128), jnp.int32),
            jax.ShapeDtypeStruct((EROWS, 128), jnp.int32),
            jax.ShapeDtypeStruct((EROWS, NT), jnp.int32),
        ],
    )(src2d, dst2d)


EPT = E // NT
CHS = 80


@functools.partial(
    pl.kernel,
    out_type=jax.ShapeDtypeStruct((B * NT * CAPP,), jnp.int32),
    mesh=_mesh,
    compiler_params=_sc_params,
    scratch_types=[
        pltpu.VMEM((CHS,), jnp.int32),
        pltpu.VMEM((CHS,), jnp.int32),
        pltpu.VMEM((CAPP,), jnp.int32),
        pltpu.VMEM_SHARED((NT * CAPP,), jnp.int32),
        pltpu.SemaphoreType.DMA,
    ],
)
def _scatlist(tgt_hbm, pk_hbm, lists_hbm, tgtv, pkv, fill, listsS, sem):
    c = lax.axis_index("c")
    t = lax.axis_index("s")

    dummy = jnp.full((16,), c * N * 1024 + NR, jnp.int32)

    def fl(i, _):
        fill[pl.ds(i * 16, 16)] = dummy
        return 0

    lax.fori_loop(0, CAPP // 16, fl, 0)
    pltpu.sync_copy(fill, listsS.at[pl.ds(t * CAPP, CAPP)])
    plsc.subcore_barrier()

    def chunk(j, _):
        pltpu.sync_copy(tgt_hbm.at[pl.ds(t * EPT + j * CHS, CHS)], tgtv)
        pltpu.sync_copy(pk_hbm.at[pl.ds(c * E + t * EPT + j * CHS, CHS)], pkv)
        pltpu.sync_copy(pkv, listsS.at[tgtv])
        return 0

    lax.fori_loop(0, EPT // CHS, chunk, 0)
    plsc.subcore_barrier()
    pltpu.sync_copy(listsS.at[pl.ds(t * CAPP, CAPP)],
                    lists_hbm.at[pl.ds((c * NT + t) * CAPP, CAPP)])


def _make_segsum(width, ch):
    @functools.partial(
        pl.kernel,
        out_type=jax.ShapeDtypeStruct((B * N, width), jnp.float32),
        mesh=_mesh,
        compiler_params=_sc_params,
        scratch_types=[
            pltpu.VMEM((ch,), jnp.int32),
            pltpu.VMEM((ch,), jnp.int32),
            pltpu.VMEM((ch,), jnp.int32),
            pltpu.VMEM((ch,), jnp.int32),
            pltpu.VMEM((ch, width), jnp.float32),
            pltpu.VMEM((ch, width), jnp.float32),
            pltpu.VMEM((16,), jnp.int32),
            pltpu.VMEM((NR + 16, width), jnp.float32),
            pltpu.SemaphoreType.DMA,
            pltpu.SemaphoreType.DMA,
        ],
    )
    def seg(lists_hbm, cnt_hbm, table_hbm, out_hbm,
            pk0, pk1, idx0, idx1, rows0, rows1, cntv, acc, sem0, sem1):
        c = lax.axis_index("c")
        t = lax.axis_index("s")
        base = (c * NT + t) * CAPP

        zero = jnp.zeros((16,), jnp.float32)

        def zr(i, _):
            for v in range(width // 16):
                acc[i, pl.ds(v * 16, 16)] = zero
            return 0

        lax.fori_loop(0, NR + 16, zr, 0)

        pltpu.sync_copy(cnt_hbm.at[pl.ds((c * NT + t) * 16, 16)], cntv)
        nrounds = cntv[pl.ds(0, 16)][0]

        def load_unp(r, pkb, idxb):
            pltpu.sync_copy(lists_hbm.at[pl.ds(base + r * ch, ch)], pkb)

            def unp(j, _):
                idxb[pl.ds(j * 16, 16)] = (
                    lax.shift_right_logical(pkb[pl.ds(j * 16, 16)], 10))
                return 0

            lax.fori_loop(0, ch // 16, unp, 0)

        def grp_all(pkb, rowsb):
            def grp(j, _):
                dv = pkb[pl.ds(j * 16, 16)] & 1023
                for k in range(16):
                    d = dv[k]
                    for v in range(width // 16):
                        plsc.addupdate(acc.at[d, pl.ds(v * 16, 16)],
                                       rowsb[j * 16 + k, pl.ds(v * 16, 16)])
                return 0

            lax.fori_loop(0, ch // 16, grp, 0)

        @pl.when(nrounds > 0)
        def _():
            load_unp(0, pk0, idx0)
            pltpu.async_copy(table_hbm.at[idx0], rows0, sem0)

        def pair(i, _):
            ra = 2 * i
            rb = 2 * i + 1

            @pl.when(rb < nrounds)
            def _():
                load_unp(rb, pk1, idx1)

            pltpu.make_async_copy(table_hbm.at[idx0], rows0, sem0).wait()

            @pl.when(rb < nrounds)
            def _():
                pltpu.async_copy(table_hbm.at[idx1], rows1, sem1)

            grp_all(pk0, rows0)

            @pl.when(ra + 2 < nrounds)
            def _():
                load_unp(ra + 2, pk0, idx0)
                pltpu.async_copy(table_hbm.at[idx0], rows0, sem0)

            @pl.when(rb < nrounds)
            def _():
                pltpu.make_async_copy(table_hbm.at[idx1], rows1, sem1).wait()
                grp_all(pk1, rows1)

            return 0

        lax.fori_loop(0, (nrounds + 1) // 2, pair, 0)
        pltpu.sync_copy(acc.at[pl.ds(0, NR)],
                        out_hbm.at[pl.ds(c * N + t * NR, NR)])

    return seg


CH128 = 160
CH16 = 512
_segsum128 = _make_segsum(F_IN, CH128)
_segsum16 = _make_segsum(H, CH16)


@functools.partial(
    pl.kernel,
    out_type=jax.ShapeDtypeStruct((B, KP, H), jnp.float32),
    mesh=_mesh,
    compiler_params=_sc_params,
    scratch_types=[
        pltpu.VMEM((KPT,), jnp.int32),
        pltpu.VMEM((KPT,), jnp.float32),
        pltpu.VMEM((KPT, H), jnp.float32),
        pltpu.VMEM((KPT, H), jnp.float32),
        pltpu.SemaphoreType.DMA,
    ],
)
def _pool(h_hbm, perm_hbm, tops_hbm, out_hbm, permv, sv, rows, orows, sem):
    c = lax.axis_index("c")
    s = lax.axis_index("s")
    pltpu.sync_copy(perm_hbm.at[c, pl.ds(s * KPT, KPT)], permv)
    pltpu.sync_copy(tops_hbm.at[c, pl.ds(s * KPT, KPT)], sv)
    pltpu.async_copy(h_hbm.at[permv], rows, sem).wait()

    def body(j, carry):
        svc = sv[pl.ds(j * 16, 16)]
        for k in range(16):
            r = j * 16 + k
            orows[r] = rows[r] * svc[k]
        return carry

    lax.fori_loop(0, KPT // 16, body, 0)
    pltpu.sync_copy(orows, out_hbm.at[c, pl.ds(s * KPT, KPT)])


def _hid_body(agg_ref, x_ref, wrel_ref, wroot_ref, brel_ref, h_ref):
    d1 = lax.dot_general(agg_ref[...], wrel_ref[...], (((1,), (1,)), ((), ())),
                         preferred_element_type=jnp.float32)
    t1 = d1 + brel_ref[...]
    d2 = lax.dot_general(x_ref[...], wroot_ref[...], (((1,), (1,)), ((), ())),
                         preferred_element_type=jnp.float32)
    h_ref[...] = jnp.maximum(t1 + d2, 0.0)


def _hid(agg, x, W_rel, W_root, brel):
    m_blk = 4000
    return pl.pallas_call(
        _hid_body,
        grid=(B * N // m_blk,),
        in_specs=[
            pl.BlockSpec((m_blk, F_IN), lambda i: (i, 0)),
            pl.BlockSpec((m_blk, F_IN), lambda i: (i, 0)),
            pl.BlockSpec((H, F_IN), lambda i: (0, 0)),
            pl.BlockSpec((H, F_IN), lambda i: (0, 0)),
            pl.BlockSpec((1, H), lambda i: (0, 0)),
        ],
        out_specs=pl.BlockSpec((m_blk, H), lambda i: (i, 0)),
        out_shape=jax.ShapeDtypeStruct((B * N, H), jnp.float32),
    )(agg, x, W_rel, W_root, brel)


def _score_body(agg2_ref, h_ref, wsrel_ref, wsroot_ref, bs_ref, o_ref):
    d1 = lax.dot_general(agg2_ref[...], wsrel_ref[...], (((1,), (0,)), ((), ())),
                         preferred_element_type=jnp.float32)
    t1 = d1 + bs_ref[0, 0]
    d2 = lax.dot_general(h_ref[...], wsroot_ref[...], (((1,), (0,)), ((), ())),
                         preferred_element_type=jnp.float32)
    o_ref[...] = jnp.tanh(t1 + d2)


def _score(agg2, h, Ws_rel, Ws_root, bs):
    wr = jnp.pad(Ws_rel.T, ((0, 0), (0, 127)))
    wo = jnp.pad(Ws_root.T, ((0, 0), (0, 127)))
    out = pl.pallas_call(
        _score_body,
        out_shape=jax.ShapeDtypeStruct((B * N, 128), jnp.float32),
    )(agg2, h, wr, wo, bs)
    return out[:, 0]


_FC_CHUNK = 6400
_FC_STEPS = (K * H) // _FC_CHUNK


def _fc_body(pf_ref, w1_ref, b1_ref, w2_ref, b2_ref, w3_ref, b3_ref,
             o_ref, acc_ref):
    k = pl.program_id(0)

    @pl.when(k == 0)
    def _():
        acc_ref[...] = jnp.zeros_like(acc_ref)

    acc_ref[...] += lax.dot_general(
        pf_ref[...], w1_ref[...], (((1,), (1,)), ((), ())),
        precision=lax.Precision.HIGHEST,
        preferred_element_type=jnp.float32)

    @pl.when(k == _FC_STEPS - 1)
    def _():
        xf = jnp.maximum(acc_ref[...] + b1_ref[...], 0.0)
        xf = jnp.maximum(
            lax.dot_general(xf, w2_ref[...], (((1,), (1,)), ((), ())),
                            precision=lax.Precision.HIGHEST,
                            preferred_element_type=jnp.float32)
            + b2_ref[...], 0.0)
        o_ref[...] = (jnp.sum(xf * w3_ref[...], axis=1, keepdims=True)
                      + b3_ref[...])


def _fc(pf, W1, b1, W2, b2, W3, b3):
    return pl.pallas_call(
        _fc_body,
        grid=(_FC_STEPS,),
        in_specs=[
            pl.BlockSpec((B, _FC_CHUNK), lambda k: (0, k)),
            pl.BlockSpec((256, _FC_CHUNK), lambda k: (0, k)),
            pl.BlockSpec((1, 256), lambda k: (0, 0)),
            pl.BlockSpec((64, 256), lambda k: (0, 0)),
            pl.BlockSpec((1, 64), lambda k: (0, 0)),
            pl.BlockSpec((1, 64), lambda k: (0, 0)),
            pl.BlockSpec((1, 1), lambda k: (0, 0)),
        ],
        out_specs=pl.BlockSpec((B, 1), lambda k: (0, 0)),
        out_shape=jax.ShapeDtypeStruct((B, 1), jnp.float32),
        scratch_shapes=[pltpu.VMEM((B, 256), jnp.float32)],
    )(pf, W1, b1, W2, b2, W3, b3)


def kernel(data, edge_index, W_rel, b_rel, W_root, Ws_rel, bs_rel, Ws_root,
           W1, b1, W2, b2, W3, b3):
    x = data.reshape(B * N, F_IN)
    src2d = edge_index[0].reshape(EROWS, 128)
    dst2d = edge_index[1].reshape(EROWS, 128)

    tgt, pk, cnt_rows = _route(src2d, dst2d)
    tgt = tgt.reshape(E)
    pk = pk.reshape(E)
    pk2 = jnp.concatenate([pk, pk + N * 1024])
    cnts = cnt_rows[EROWS - 1]
    nr128 = (cnts + CH128 - 1) // CH128
    nr16 = (cnts + CH16 - 1) // CH16
    nr128_s = jnp.broadcast_to(nr128[None, :, None], (B, NT, 16)).reshape(-1)
    nr16_s = jnp.broadcast_to(nr16[None, :, None], (B, NT, 16)).reshape(-1)

    lists = _scatlist(tgt, pk2)
    agg = _segsum128(lists, nr128_s, x)
    h = _hid(agg, x, W_rel, W_root, b_rel.reshape(1, H))
    agg2 = _segsum16(lists, nr16_s, h)
    score = _score(agg2, h, Ws_rel, Ws_root, bs_rel.reshape(1, 1))
    score = score.reshape(B, N)

    top_s, perm = lax.top_k(score, K)
    offs = jnp.arange(B, dtype=jnp.int32) * N
    perm_g = jnp.pad(perm + offs[:, None], ((0, 0), (0, KP - K)))
    tops_p = jnp.pad(top_s, ((0, 0), (0, KP - K)))

    pooled = _pool(h, perm_g, tops_p)
    pf = pooled[:, :K, :].reshape(B, K * H)

    return _fc(pf, W1, b1.reshape(1, 256), W2, b2.reshape(1, 64),
               W3, b3.reshape(1, 1))


# speedup vs baseline: 4.1755x; 1.1333x over previous
import functools

import jax
import jax.numpy as jnp
from jax import lax
from jax.experimental import pallas as pl
from jax.experimental.pallas import tpu as pltpu
from jax.experimental.pallas import tpu_sc as plsc

N = 10000
F_IN = 128
H = 16
B = 2
E = 160000
K = 4000

NT = 16
NR = N // NT
CAP = 16384
CAPP = CAP + 256
KP = 4096
KPT = KP // 16

_mesh = plsc.VectorSubcoreMesh(core_axis_name="c", subcore_axis_name="s")
_sc_params = pltpu.CompilerParams(use_tc_tiling_on_sc=False)


EROWS = E // 128


def _route_body(src_ref, dst_ref, tgt_ref, pk_ref, cnt_ref):
    dst = dst_ref[...]
    src = src_ref[...]
    rid = dst // NR
    dl = dst - rid * NR
    wrow = jnp.zeros_like(dst)
    rows = []
    l0 = lax.broadcasted_iota(jnp.int32, (128, 128), 0)
    l1 = lax.broadcasted_iota(jnp.int32, (128, 128), 1)
    ltl = (l0 < l1).astype(jnp.float32)
    for r in range(NT):
        mr = rid == r
        mrf = mr.astype(jnp.float32)
        cr = lax.dot_general(mrf, ltl, (((1,), (0,)), ((), ())),
                             preferred_element_type=jnp.float32)
        wrow = wrow + jnp.where(mr, cr.astype(jnp.int32), 0)
        rows.append(jnp.sum(mrf, axis=1, keepdims=True))
    rowsum = jnp.concatenate(rows, axis=1).astype(jnp.float32)
    i0 = lax.broadcasted_iota(jnp.int32, (EROWS, EROWS), 0)
    i1 = lax.broadcasted_iota(jnp.int32, (EROWS, EROWS), 1)
    lt = (i1 < i0).astype(jnp.float32)
    rowpref = lax.dot_general(lt, rowsum, (((1,), (0,)), ((), ())),
                              preferred_element_type=jnp.float32)
    base = jnp.take_along_axis(rowpref.astype(jnp.int32), rid, axis=1)
    pos = base + wrow
    tgt_ref[...] = rid * CAPP + pos
    pk_ref[...] = src * 1024 + dl
    cnt_ref[...] = (rowpref + rowsum).astype(jnp.int32)


def _route(src2d, dst2d):
    return pl.pallas_call(
        _route_body,
        out_shape=[
            jax.ShapeDtypeStruct((EROWS, 128), jnp.int32),
            jax.ShapeDtypeStruct((EROWS, 128), jnp.int32),
            jax.ShapeDtypeStruct((EROWS, NT), jnp.int32),
        ],
    )(src2d, dst2d)


EPT = E // NT
CHS = 80


@functools.partial(
    pl.kernel,
    out_type=jax.ShapeDtypeStruct((B * NT * CAPP,), jnp.int32),
    mesh=_mesh,
    compiler_params=_sc_params,
    scratch_types=[
        pltpu.VMEM((25, CHS), jnp.int32),
        pltpu.VMEM((25, CHS), jnp.int32),
        pltpu.VMEM((CAPP,), jnp.int32),
        pltpu.VMEM_SHARED((NT * CAPP,), jnp.int32),
        pltpu.SemaphoreType.DMA,
    ],
)
def _scatlist(tgt_hbm, pk_hbm, lists_hbm, tgtv, pkv, fill, listsS, sem):
    c = lax.axis_index("c")
    t = lax.axis_index("s")

    dummy = jnp.full((16,), c * N * 1024 + NR, jnp.int32)

    def fl(i, _):
        fill[pl.ds(i * 16, 16)] = dummy
        return 0

    lax.fori_loop(0, CAPP // 16, fl, 0)
    pltpu.sync_copy(fill, listsS.at[pl.ds(t * CAPP, CAPP)])
    plsc.subcore_barrier()

    rows_pt = EPT // CHS
    rpslab = 25

    def slab(j, _):
        rbase = t * rows_pt + j * rpslab
        pltpu.sync_copy(tgt_hbm.at[pl.ds(rbase, rpslab)], tgtv)
        pltpu.sync_copy(pk_hbm.at[pl.ds(c * (E // CHS) + rbase, rpslab)], pkv)

        def sc80(k, _):
            pltpu.sync_copy(pkv.at[k], listsS.at[tgtv.at[k]])
            return 0

        lax.fori_loop(0, rpslab, sc80, 0)
        return 0

    lax.fori_loop(0, rows_pt // rpslab, slab, 0)
    plsc.subcore_barrier()
    pltpu.sync_copy(listsS.at[pl.ds(t * CAPP, CAPP)],
                    lists_hbm.at[pl.ds((c * NT + t) * CAPP, CAPP)])


def _make_segsum(width, ch):
    @functools.partial(
        pl.kernel,
        out_type=jax.ShapeDtypeStruct((B * N, width), jnp.float32),
        mesh=_mesh,
        compiler_params=_sc_params,
        scratch_types=[
            pltpu.VMEM((ch,), jnp.int32),
            pltpu.VMEM((ch,), jnp.int32),
            pltpu.VMEM((ch,), jnp.int32),
            pltpu.VMEM((ch,), jnp.int32),
            pltpu.VMEM((ch, width), jnp.float32),
            pltpu.VMEM((ch, width), jnp.float32),
            pltpu.VMEM((16,), jnp.int32),
            pltpu.VMEM((NR + 16, width), jnp.float32),
            pltpu.SemaphoreType.DMA,
            pltpu.SemaphoreType.DMA,
        ],
    )
    def seg(lists_hbm, cnt_hbm, table_hbm, out_hbm,
            pk0, pk1, idx0, idx1, rows0, rows1, cntv, acc, sem0, sem1):
        c = lax.axis_index("c")
        t = lax.axis_index("s")
        base = (c * NT + t) * CAPP

        zero = jnp.zeros((16,), jnp.float32)

        def zr(i, _):
            for v in range(width // 16):
                acc[i, pl.ds(v * 16, 16)] = zero
            return 0

        lax.fori_loop(0, NR + 16, zr, 0)

        pltpu.sync_copy(cnt_hbm.at[pl.ds((c * NT + t) * 16, 16)], cntv)
        nrounds = cntv[pl.ds(0, 16)][0]

        def load_unp(r, pkb, idxb):
            pltpu.sync_copy(lists_hbm.at[pl.ds(base + r * ch, ch)], pkb)

            def unp(j, _):
                idxb[pl.ds(j * 16, 16)] = (
                    lax.shift_right_logical(pkb[pl.ds(j * 16, 16)], 10))
                return 0

            lax.fori_loop(0, ch // 16, unp, 0)

        def grp_all(pkb, rowsb):
            def grp(j, _):
                dv = pkb[pl.ds(j * 16, 16)] & 1023
                for k in range(16):
                    d = dv[k]
                    for v in range(width // 16):
                        plsc.addupdate(acc.at[d, pl.ds(v * 16, 16)],
                                       rowsb[j * 16 + k, pl.ds(v * 16, 16)])
                return 0

            lax.fori_loop(0, ch // 16, grp, 0)

        @pl.when(nrounds > 0)
        def _():
            load_unp(0, pk0, idx0)
            pltpu.async_copy(table_hbm.at[idx0], rows0, sem0)

        def pair(i, _):
            ra = 2 * i
            rb = 2 * i + 1

            @pl.when(rb < nrounds)
            def _():
                load_unp(rb, pk1, idx1)

            pltpu.make_async_copy(table_hbm.at[idx0], rows0, sem0).wait()

            @pl.when(rb < nrounds)
            def _():
                pltpu.async_copy(table_hbm.at[idx1], rows1, sem1)

            grp_all(pk0, rows0)

            @pl.when(ra + 2 < nrounds)
            def _():
                load_unp(ra + 2, pk0, idx0)
                pltpu.async_copy(table_hbm.at[idx0], rows0, sem0)

            @pl.when(rb < nrounds)
            def _():
                pltpu.make_async_copy(table_hbm.at[idx1], rows1, sem1).wait()
                grp_all(pk1, rows1)

            return 0

        lax.fori_loop(0, (nrounds + 1) // 2, pair, 0)
        pltpu.sync_copy(acc.at[pl.ds(0, NR)],
                        out_hbm.at[pl.ds(c * N + t * NR, NR)])

    return seg


CH128 = 160
CH16 = 512
_segsum128 = _make_segsum(F_IN, CH128)
_segsum16 = _make_segsum(H, CH16)


@functools.partial(
    pl.kernel,
    out_type=jax.ShapeDtypeStruct((B, KP, H), jnp.float32),
    mesh=_mesh,
    compiler_params=_sc_params,
    scratch_types=[
        pltpu.VMEM((KPT,), jnp.int32),
        pltpu.VMEM((KPT,), jnp.float32),
        pltpu.VMEM((KPT, H), jnp.float32),
        pltpu.VMEM((KPT, H), jnp.float32),
        pltpu.SemaphoreType.DMA,
    ],
)
def _pool(h_hbm, perm_hbm, tops_hbm, out_hbm, permv, sv, rows, orows, sem):
    c = lax.axis_index("c")
    s = lax.axis_index("s")
    pltpu.sync_copy(perm_hbm.at[c, pl.ds(s * KPT, KPT)], permv)
    pltpu.sync_copy(tops_hbm.at[c, pl.ds(s * KPT, KPT)], sv)
    pltpu.async_copy(h_hbm.at[permv], rows, sem).wait()

    def body(j, carry):
        svc = sv[pl.ds(j * 16, 16)]
        for k in range(16):
            r = j * 16 + k
            orows[r] = rows[r] * svc[k]
        return carry

    lax.fori_loop(0, KPT // 16, body, 0)
    pltpu.sync_copy(orows, out_hbm.at[c, pl.ds(s * KPT, KPT)])


def _hid_body(agg_ref, x_ref, wrel_ref, wroot_ref, brel_ref, h_ref):
    d1 = lax.dot_general(agg_ref[...], wrel_ref[...], (((1,), (1,)), ((), ())),
                         preferred_element_type=jnp.float32)
    t1 = d1 + brel_ref[...]
    d2 = lax.dot_general(x_ref[...], wroot_ref[...], (((1,), (1,)), ((), ())),
                         preferred_element_type=jnp.float32)
    h_ref[...] = jnp.maximum(t1 + d2, 0.0)


def _hid(agg, x, W_rel, W_root, brel):
    m_blk = 4000
    return pl.pallas_call(
        _hid_body,
        grid=(B * N // m_blk,),
        in_specs=[
            pl.BlockSpec((m_blk, F_IN), lambda i: (i, 0)),
            pl.BlockSpec((m_blk, F_IN), lambda i: (i, 0)),
            pl.BlockSpec((H, F_IN), lambda i: (0, 0)),
            pl.BlockSpec((H, F_IN), lambda i: (0, 0)),
            pl.BlockSpec((1, H), lambda i: (0, 0)),
        ],
        out_specs=pl.BlockSpec((m_blk, H), lambda i: (i, 0)),
        out_shape=jax.ShapeDtypeStruct((B * N, H), jnp.float32),
    )(agg, x, W_rel, W_root, brel)


def _score_body(agg2_ref, h_ref, wsrel_ref, wsroot_ref, bs_ref, o_ref):
    d1 = lax.dot_general(agg2_ref[...], wsrel_ref[...], (((1,), (0,)), ((), ())),
                         preferred_element_type=jnp.float32)
    t1 = d1 + bs_ref[0, 0]
    d2 = lax.dot_general(h_ref[...], wsroot_ref[...], (((1,), (0,)), ((), ())),
                         preferred_element_type=jnp.float32)
    o_ref[...] = jnp.tanh(t1 + d2)


def _score(agg2, h, Ws_rel, Ws_root, bs):
    wr = jnp.pad(Ws_rel.T, ((0, 0), (0, 127)))
    wo = jnp.pad(Ws_root.T, ((0, 0), (0, 127)))
    out = pl.pallas_call(
        _score_body,
        out_shape=jax.ShapeDtypeStruct((B * N, 128), jnp.float32),
    )(agg2, h, wr, wo, bs)
    return out[:, 0]


_FC_CHUNK = 6400
_FC_STEPS = (K * H) // _FC_CHUNK


def _fc_body(pf_ref, w1_ref, b1_ref, w2_ref, b2_ref, w3_ref, b3_ref,
             o_ref, acc_ref):
    k = pl.program_id(0)

    @pl.when(k == 0)
    def _():
        acc_ref[...] = jnp.zeros_like(acc_ref)

    acc_ref[...] += lax.dot_general(
        pf_ref[...], w1_ref[...], (((1,), (1,)), ((), ())),
        precision=lax.Precision.HIGHEST,
        preferred_element_type=jnp.float32)

    @pl.when(k == _FC_STEPS - 1)
    def _():
        xf = jnp.maximum(acc_ref[...] + b1_ref[...], 0.0)
        xf = jnp.maximum(
            lax.dot_general(xf, w2_ref[...], (((1,), (1,)), ((), ())),
                            precision=lax.Precision.HIGHEST,
                            preferred_element_type=jnp.float32)
            + b2_ref[...], 0.0)
        o_ref[...] = (jnp.sum(xf * w3_ref[...], axis=1, keepdims=True)
                      + b3_ref[...])


def _fc(pf, W1, b1, W2, b2, W3, b3):
    return pl.pallas_call(
        _fc_body,
        grid=(_FC_STEPS,),
        in_specs=[
            pl.BlockSpec((B, _FC_CHUNK), lambda k: (0, k)),
            pl.BlockSpec((256, _FC_CHUNK), lambda k: (0, k)),
            pl.BlockSpec((1, 256), lambda k: (0, 0)),
            pl.BlockSpec((64, 256), lambda k: (0, 0)),
            pl.BlockSpec((1, 64), lambda k: (0, 0)),
            pl.BlockSpec((1, 64), lambda k: (0, 0)),
            pl.BlockSpec((1, 1), lambda k: (0, 0)),
        ],
        out_specs=pl.BlockSpec((B, 1), lambda k: (0, 0)),
        out_shape=jax.ShapeDtypeStruct((B, 1), jnp.float32),
        scratch_shapes=[pltpu.VMEM((B, 256), jnp.float32)],
    )(pf, W1, b1, W2, b2, W3, b3)


def kernel(data, edge_index, W_rel, b_rel, W_root, Ws_rel, bs_rel, Ws_root,
           W1, b1, W2, b2, W3, b3):
    x = data.reshape(B * N, F_IN)
    src2d = edge_index[0].reshape(EROWS, 128)
    dst2d = edge_index[1].reshape(EROWS, 128)

    tgt, pk, cnt_rows = _route(src2d, dst2d)
    tgt = tgt.reshape(E)
    pk = pk.reshape(E)
    pk2 = jnp.concatenate([pk, pk + N * 1024])
    cnts = cnt_rows[EROWS - 1]
    nr128 = (cnts + CH128 - 1) // CH128
    nr16 = (cnts + CH16 - 1) // CH16
    nr128_s = jnp.broadcast_to(nr128[None, :, None], (B, NT, 16)).reshape(-1)
    nr16_s = jnp.broadcast_to(nr16[None, :, None], (B, NT, 16)).reshape(-1)

    lists = _scatlist(tgt.reshape(E // CHS, CHS),
                      pk2.reshape(B * E // CHS, CHS))
    agg = _segsum128(lists, nr128_s, x)
    h = _hid(agg, x, W_rel, W_root, b_rel.reshape(1, H))
    agg2 = _segsum16(lists, nr16_s, h)
    score = _score(agg2, h, Ws_rel, Ws_root, bs_rel.reshape(1, 1))
    score = score.reshape(B, N)

    top_s, perm = lax.top_k(score, K)
    offs = jnp.arange(B, dtype=jnp.int32) * N
    perm_g = jnp.pad(perm + offs[:, None], ((0, 0), (0, KP - K)))
    tops_p = jnp.pad(top_s, ((0, 0), (0, KP - K)))

    pooled = _pool(h, perm_g, tops_p)
    pf = pooled[:, :K, :].reshape(B, K * H)

    return _fc(pf, W1, b1.reshape(1, 256), W2, b2.reshape(1, 64),
               W3, b3.reshape(1, 1))
